# Initial kernel scaffold; baseline (speedup 1.0000x reference)
import jax
import jax.numpy as jnp
from jax.experimental import pallas as pl


def kernel(x, edge_index, W1, b1, W2, b2):
    raise NotImplementedError("write your pallas kernel here")


# speedup vs baseline: 4.8019x; 4.8019x over previous
import functools

import jax
import jax.numpy as jnp
from jax import lax
from jax.experimental import pallas as pl
from jax.experimental.pallas import tpu as pltpu
from jax.experimental.pallas import tpu_sc as plsc

NC = 2
NS = 16
NW = NC * NS
CHUNK = 128


def _mm_body(x_ref, w_ref, o_ref):
    o_ref[...] = jnp.dot(x_ref[...], w_ref[...],
                         preferred_element_type=jnp.float32)


def _matmul(x, w):
    n, _ = x.shape
    _, dout = w.shape
    return pl.pallas_call(
        _mm_body,
        out_shape=jax.ShapeDtypeStruct((n, dout), jnp.float32),
    )(x, w)


def _combine_relu_mm_body(n, p_ref, b_ref, w_ref, o_ref):
    h = p_ref[0, :n] + p_ref[1, :n] + b_ref[...]
    h = jnp.maximum(h, 0.0)
    o_ref[...] = jnp.dot(h, w_ref[...], preferred_element_type=jnp.float32)


def _combine_relu_mm(parts, b, w, n):
    dout = w.shape[1]
    return pl.pallas_call(
        functools.partial(_combine_relu_mm_body, n),
        out_shape=jax.ShapeDtypeStruct((n, dout), jnp.float32),
    )(parts, b, w)


def _combine_bias_body(n, p_ref, b_ref, o_ref):
    o_ref[...] = p_ref[0, :n] + p_ref[1, :n] + b_ref[...]


def _combine_bias(parts, b, n):
    _, _, d = parts.shape
    return pl.pallas_call(
        functools.partial(_combine_bias_body, n),
        out_shape=jax.ShapeDtypeStruct((n, d), jnp.float32),
    )(parts, b)


def _make_aggregate(n_pad, d, chunks):
    mesh = plsc.VectorSubcoreMesh(core_axis_name="c", subcore_axis_name="s")
    rows_per_sub = n_pad // NS

    @functools.partial(
        pl.kernel,
        mesh=mesh,
        out_type=jax.ShapeDtypeStruct((NC, n_pad, d), jnp.float32),
        scratch_types=[
            pltpu.VMEM((chunks, CHUNK), jnp.int32),
            pltpu.VMEM((chunks, CHUNK), jnp.int32),
            pltpu.VMEM((CHUNK, d), jnp.float32),
            pltpu.VMEM_SHARED((n_pad, d), jnp.float32),
            pltpu.SemaphoreType.DMA,
        ],
    )
    def aggregate(h_hbm, src_hbm, dst_hbm, zeros_hbm, out_hbm,
                  src_v, dst_v, rows_v, acc, sem):
        c = lax.axis_index("c")
        s = lax.axis_index("s")
        wid = c * NS + s
        row0 = s * rows_per_sub
        pltpu.sync_copy(zeros_hbm.at[pl.ds(row0, rows_per_sub)],
                        acc.at[pl.ds(row0, rows_per_sub)])
        pltpu.sync_copy(src_hbm.at[wid], src_v)
        pltpu.sync_copy(dst_hbm.at[wid], dst_v)
        plsc.subcore_barrier()

        def body(j, carry):
            pltpu.async_copy(h_hbm.at[src_v.at[j]], rows_v, sem).wait()
            pltpu.sync_copy(rows_v, acc.at[dst_v.at[j]], add=True)
            return carry

        lax.fori_loop(0, chunks, body, 0)
        plsc.subcore_barrier()
        pltpu.sync_copy(acc.at[pl.ds(row0, rows_per_sub)],
                        out_hbm.at[c, pl.ds(row0, rows_per_sub)])

    return aggregate


def kernel(x, edge_index, W1, b1, W2, b2):
    n, d = x.shape
    e = edge_index.shape[1]

    chunks = -(-e // (NW * CHUNK))
    e_pad = NW * chunks * CHUNK
    n_pad = -(-(n + 1) // (NS * 8)) * (NS * 8)
    src = jnp.concatenate(
        [edge_index[0], jnp.zeros((e_pad - e,), jnp.int32)]).reshape(
            NW, chunks, CHUNK)
    dst = jnp.concatenate(
        [edge_index[1], jnp.full((e_pad - e,), n, jnp.int32)]).reshape(
            NW, chunks, CHUNK)
    zeros = jnp.zeros((n_pad, d), jnp.float32)

    aggregate = _make_aggregate(n_pad, d, chunks)

    h1 = _matmul(x, W1)
    p1 = aggregate(h1, src, dst, zeros)
    h2 = _combine_relu_mm(p1, b1, W2, n)
    p2 = aggregate(h2, src, dst, zeros)
    return _combine_bias(p2, b2, n)
